# baseline (speedup 1.0000x reference)
import dataclasses
import functools

import jax
import jax.numpy as jnp
from jax import lax
from jax.experimental import pallas as pl
from jax.experimental.pallas import tpu as pltpu
from jax.experimental.pallas import tpu_sc as plsc

N_TOK = 8192
D_IN = 1024
H = 256
T = 8
RBLK = 512
NRBLK = N_TOK // RBLK
MBLK = 512
PBLK = N_TOK // MBLK + T
GRID_MLP = PBLK + 1
PAD_ROWS = GRID_MLP * MBLK
DUMP = PBLK * MBLK
NW = 32
TPW = N_TOK // NW


def _router(state, tempers, done, emb_ref, rw1_ref, rb1_ref, rw2_ref,
            rb2_ref, tril_ref):
    emb = jnp.zeros((RBLK, 4), jnp.float32)
    for t in range(T):
        emb = jnp.where(tempers == t, emb_ref[t:t + 1, :], emb)
    rh = (jnp.dot(state, rw1_ref[0:H, :])
          + jnp.dot(emb, rw1_ref[H:H + 4, :])
          + rb1_ref[...])
    rh = jnp.maximum(rh, 0.0)
    logits = jnp.dot(rh, rw2_ref[...]) + rb2_ref[...]
    mx = jnp.max(logits, axis=1, keepdims=True)
    cand = jnp.where(logits == mx,
                     jax.lax.broadcasted_iota(jnp.int32, (RBLK, T + 1), 1),
                     T + 1)
    action = jnp.min(cand, axis=1, keepdims=True)
    bins = jnp.where(done, T, action)
    oh9 = (bins == jax.lax.broadcasted_iota(jnp.int32, (RBLK, 128), 1)
           ).astype(jnp.float32)
    cum = jnp.dot(tril_ref[...], oh9)
    tot = jnp.sum(oh9, axis=0, keepdims=True)
    return bins, oh9, cum, tot


def _rank_from(oh9, cum, run_row):
    rank_f = jnp.sum(oh9 * (run_row + cum), axis=1, keepdims=True) - 1.0
    return rank_f.astype(jnp.int32)


def _emit_tables(cnt, triu8_ref, offs_out, cumb_out):
    nb = jnp.floor((cnt + (MBLK - 1)) * (1.0 / MBLK))
    cumb = jnp.dot(nb, triu8_ref[...])
    offs = (cumb - nb) * MBLK
    lane = jax.lax.broadcasted_iota(jnp.int32, (1, 128), 1)
    offs_out[...] = jnp.where(lane < T, offs, float(DUMP)).astype(jnp.int32)
    cumb_out[...] = cumb.astype(jnp.int32)


def _k1_body(x_ref, w_in_ref, b_in_ref, emb_ref, rw1_ref, rb1_ref, rw2_ref,
             rb2_ref, t0_ref, tril_ref, triu8_ref,
             state_out, bins_out, rank_out, offs_out, cumb_out, run_ref):
    i = pl.program_id(0)

    @pl.when(i == 0)
    def _():
        run_ref[...] = jnp.zeros((8, 128), jnp.float32)

    state = jnp.dot(x_ref[...], w_in_ref[...]) + b_in_ref[...]
    state_out[...] = state
    done = jnp.zeros((RBLK, 1), jnp.bool_)
    bins, oh9, cum, tot = _router(state, t0_ref[...], done, emb_ref, rw1_ref,
                                  rb1_ref, rw2_ref, rb2_ref, tril_ref)
    bins_out[...] = bins
    run_row = run_ref[0:1, :]
    rank_out[...] = _rank_from(oh9, cum, run_row)
    run_ref[0:1, :] = run_row + tot

    @pl.when(i == NRBLK - 1)
    def _():
        _emit_tables(run_row + tot, triu8_ref, offs_out, cumb_out)


def _k3_body(g1_ref, bins1_ref, emb_ref, rw1_ref, rb1_ref, rw2_ref, rb2_ref,
             tril_ref, triu8_ref,
             bins_out, rank_out, offs_out, cumb_out, run_ref):
    i = pl.program_id(0)

    @pl.when(i == 0)
    def _():
        run_ref[...] = jnp.zeros((8, 128), jnp.float32)

    bins1 = bins1_ref[...]
    done = bins1 == T
    bins, oh9, cum, tot = _router(g1_ref[...], bins1, done, emb_ref, rw1_ref,
                                  rb1_ref, rw2_ref, rb2_ref, tril_ref)
    bins_out[...] = bins
    run_row = run_ref[0:1, :]
    rank_out[...] = _rank_from(oh9, cum, run_row)
    run_ref[0:1, :] = run_row + tot

    @pl.when(i == NRBLK - 1)
    def _():
        _emit_tables(run_row + tot, triu8_ref, offs_out, cumb_out)


def _k5_body(state1_ref, g1_ref, bins1_ref, g2_ref, bins2_ref, lng_ref,
             lnb_ref, tw_ref, tb_ref, out_ref):
    s2 = jnp.where(bins1_ref[...] < T, g1_ref[...], state1_ref[...])
    state = jnp.where(bins2_ref[...] < T, g2_ref[...], s2)
    mu = jnp.mean(state, axis=1, keepdims=True)
    var = jnp.mean((state - mu) ** 2, axis=1, keepdims=True)
    normed = (state - mu) / jnp.sqrt(var + 1e-5) * lng_ref[...] + lnb_ref[...]
    out_ref[...] = jnp.dot(normed, tw_ref[...]) + tb_ref[...]


def _mlp_body(cumb_ref, oi_ref, xin_ref, w1_ref, b1_ref, w2_ref, b2_ref,
              o_ref):
    del cumb_ref, oi_ref
    h1 = jnp.maximum(jnp.dot(xin_ref[...], w1_ref[0, 0]) + b1_ref[0, 0], 0.0)
    o_ref[...] = jnp.maximum(jnp.dot(h1, w2_ref[0, 0]) + b2_ref[0, 0],
                             0.0) * 1.01


def _whole(shape):
    return pl.BlockSpec(shape, lambda i: tuple(0 for _ in shape))


@functools.lru_cache(maxsize=1)
def _sc_mesh():
    return plsc.VectorSubcoreMesh(core_axis_name="c", subcore_axis_name="s")


@functools.lru_cache(maxsize=1)
def _sc_params():
    cp = pltpu.CompilerParams()
    if "needs_layout_passes" in pltpu.CompilerParams.__dataclass_fields__:
        cp = dataclasses.replace(cp, needs_layout_passes=False)
    return cp


def _sc_scatter(bins2d, rank2d, offs16, state):
    @functools.partial(
        pl.kernel,
        out_type=(jax.ShapeDtypeStruct((PAD_ROWS, H), jnp.float32),
                  jax.ShapeDtypeStruct((NW, 2, 128), jnp.int32)),
        mesh=_sc_mesh(),
        scratch_types=[pltpu.VMEM((2, 128), jnp.int32),
                       pltpu.VMEM((2, 128), jnp.int32),
                       pltpu.VMEM((2, 128), jnp.int32),
                       pltpu.VMEM((16,), jnp.int32),
                       pltpu.VMEM((TPW, H), jnp.float32),
                       pltpu.SemaphoreType.DMA],
        compiler_params=_sc_params(),
    )
    def scat(bins_hbm, rank_hbm, offs_hbm, state_hbm, pad_out, pos_out,
             bins_v, rank_v, pos_v, offs_v, rows_v, sem):
        wid = lax.axis_index("s") * 2 + lax.axis_index("c")
        base = wid * TPW
        cp = pltpu.async_copy(state_hbm.at[pl.ds(base, TPW)], rows_v, sem)
        pltpu.sync_copy(bins_hbm.at[pl.ds(wid * 2, 2)], bins_v)
        pltpu.sync_copy(rank_hbm.at[pl.ds(wid * 2, 2)], rank_v)
        pltpu.sync_copy(offs_hbm, offs_v)
        for cj in range(2):
            for j in range(8):
                sl = pl.ds(j * 16, 16)
                b = bins_v[cj, sl]
                r = rank_v[cj, sl]
                off = plsc.load_gather(offs_v, [b])
                pos_v[cj, sl] = jnp.where(b >= T, DUMP, off + r)
        cp.wait()
        pltpu.sync_copy(pos_v, pos_out.at[wid])
        for cj in range(2):
            pltpu.sync_copy(rows_v.at[pl.ds(cj * 128, 128)],
                            pad_out.at[pos_v.at[cj]])

    return scat(bins2d, rank2d, offs16, state)


def _sc_gather(pos, mlp_out):
    @functools.partial(
        pl.kernel,
        out_type=jax.ShapeDtypeStruct((N_TOK, H), jnp.float32),
        mesh=_sc_mesh(),
        scratch_types=[pltpu.VMEM((2, 128), jnp.int32),
                       pltpu.VMEM((TPW, H), jnp.float32),
                       pltpu.SemaphoreType.DMA],
    )
    def gath(pos_hbm, mlp_hbm, out_hbm, pos_v, rows_v, sem):
        wid = lax.axis_index("s") * 2 + lax.axis_index("c")
        base = wid * TPW
        pltpu.sync_copy(pos_hbm.at[wid], pos_v)
        cp0 = pltpu.async_copy(mlp_hbm.at[pos_v.at[0]],
                               rows_v.at[pl.ds(0, 128)], sem)
        cp1 = pltpu.async_copy(mlp_hbm.at[pos_v.at[1]],
                               rows_v.at[pl.ds(128, 128)], sem)
        cp0.wait()
        cp1.wait()
        pltpu.sync_copy(rows_v, out_hbm.at[pl.ds(base, TPW)])

    return gath(pos, mlp_out)


def _grouped_mlp(cumb8, oi, pad_buf, op_w1, op_b1, op_w2, op_b2):
    def _wmap(g, cumb_ref, oi_ref):
        e = jnp.int32(0)
        for t in range(T):
            e = e + (g >= cumb_ref[t]).astype(jnp.int32)
        e = jnp.minimum(e, T - 1)
        return (e, oi_ref[e], 0, 0)

    grid_spec = pltpu.PrefetchScalarGridSpec(
        num_scalar_prefetch=2,
        grid=(GRID_MLP,),
        in_specs=[
            pl.BlockSpec((MBLK, H), lambda g, cumb_ref, oi_ref: (g, 0)),
            pl.BlockSpec((1, 1, H, H), _wmap),
            pl.BlockSpec((1, 1, 1, H), _wmap),
            pl.BlockSpec((1, 1, H, H), _wmap),
            pl.BlockSpec((1, 1, 1, H), _wmap),
        ],
        out_specs=pl.BlockSpec((MBLK, H), lambda g, cumb_ref, oi_ref: (g, 0)),
    )
    return pl.pallas_call(
        _mlp_body,
        grid_spec=grid_spec,
        out_shape=jax.ShapeDtypeStruct((PAD_ROWS, H), jnp.float32),
    )(cumb8, oi, pad_buf, op_w1, op_b1.reshape(T, 3, 1, H), op_w2,
      op_b2.reshape(T, 3, 1, H))


def kernel(x, W_in, b_in, op_W1, op_b1, op_W2, op_b2, operator_logits,
           temper_embed, route_W1, route_b1, route_W2, route_b2, ln_g, ln_b,
           task_W, task_b, init_tempers):
    oi = jnp.argmax(operator_logits, axis=-1).astype(jnp.int32)
    t0 = init_tempers.astype(jnp.int32).reshape(N_TOK, 1)
    tril = jnp.tril(jnp.ones((RBLK, RBLK), jnp.float32))
    triu = jnp.triu(jnp.ones((128, 128), jnp.float32))
    triu8 = jnp.where(jnp.arange(128)[:, None] < T, triu, 0.0)
    rb1 = route_b1.reshape(1, H)
    rb2 = route_b2.reshape(1, T + 1)

    tok_i32 = pl.BlockSpec((RBLK, 1), lambda i: (i, 0))
    tok_f32 = pl.BlockSpec((RBLK, H), lambda i: (i, 0))
    row128 = pl.BlockSpec((1, 128), lambda i: (0, 0))
    common_w = [_whole((T, 4)), _whole((H + 4, H)), _whole((1, H)),
                _whole((H, T + 1)), _whole((1, T + 1))]
    table_outs = [jax.ShapeDtypeStruct((1, 128), jnp.int32),
                  jax.ShapeDtypeStruct((1, 128), jnp.int32)]
    tok_outs = [jax.ShapeDtypeStruct((N_TOK, 1), jnp.int32),
                jax.ShapeDtypeStruct((N_TOK, 1), jnp.int32)]

    state1, bins1, rank1, offs1, cumb1 = pl.pallas_call(
        _k1_body,
        grid=(NRBLK,),
        in_specs=[pl.BlockSpec((RBLK, D_IN), lambda i: (i, 0)),
                  _whole((D_IN, H)), _whole((1, H))] + common_w
                 + [tok_i32, _whole((RBLK, RBLK)), _whole((128, 128))],
        out_specs=[tok_f32, tok_i32, tok_i32, row128, row128],
        out_shape=[jax.ShapeDtypeStruct((N_TOK, H), jnp.float32)] + tok_outs
                  + table_outs,
        scratch_shapes=[pltpu.VMEM((8, 128), jnp.float32)],
    )(x, W_in, b_in.reshape(1, H), temper_embed, route_W1, rb1, route_W2,
      rb2, t0, tril, triu8)

    def dispatch_round(source, bins, rank, offs, cumb):
        pad_buf, pos = _sc_scatter(bins.reshape(NW * 2, 128),
                                   rank.reshape(NW * 2, 128),
                                   offs[0, :16], source)
        mlp_out = _grouped_mlp(cumb[0, :T], oi, pad_buf, op_W1, op_b1,
                               op_W2, op_b2)
        return _sc_gather(pos, mlp_out)

    g1 = dispatch_round(state1, bins1, rank1, offs1, cumb1)

    bins2, rank2, offs2, cumb2 = pl.pallas_call(
        _k3_body,
        grid=(NRBLK,),
        in_specs=[tok_f32, tok_i32] + common_w
                 + [_whole((RBLK, RBLK)), _whole((128, 128))],
        out_specs=[tok_i32, tok_i32, row128, row128],
        out_shape=tok_outs + table_outs,
        scratch_shapes=[pltpu.VMEM((8, 128), jnp.float32)],
    )(g1, bins1, temper_embed, route_W1, rb1, route_W2, rb2, tril, triu8)

    g2 = dispatch_round(g1, bins2, rank2, offs2, cumb2)

    out = pl.pallas_call(
        _k5_body,
        grid=(NRBLK,),
        in_specs=[tok_f32, tok_f32, tok_i32, tok_f32, tok_i32,
                  _whole((1, H)), _whole((1, H)), _whole((H, 10)),
                  _whole((1, 10))],
        out_specs=pl.BlockSpec((RBLK, 10), lambda i: (i, 0)),
        out_shape=jax.ShapeDtypeStruct((N_TOK, 10), jnp.float32),
    )(state1, g1, bins1, g2, bins2, ln_g.reshape(1, H), ln_b.reshape(1, H),
      task_W, task_b.reshape(1, 10))
    return out


# speedup vs baseline: 1.0032x; 1.0032x over previous
import dataclasses
import functools

import jax
import jax.numpy as jnp
from jax import lax
from jax.experimental import pallas as pl
from jax.experimental.pallas import tpu as pltpu
from jax.experimental.pallas import tpu_sc as plsc

N_TOK = 8192
D_IN = 1024
H = 256
T = 8
RBLK = 512
NRBLK = N_TOK // RBLK
MBLK = 512
PBLK = N_TOK // MBLK + T
GRID_MLP = PBLK + 1
PAD_ROWS = GRID_MLP * MBLK
DUMP = PBLK * MBLK
NW = 32
TPW = N_TOK // NW


def _router(state, tempers, done, emb_ref, rw1_ref, rb1_ref, rw2_ref,
            rb2_ref, tril_ref):
    emb = jnp.zeros((RBLK, 4), jnp.float32)
    for t in range(T):
        emb = jnp.where(tempers == t, emb_ref[t:t + 1, :], emb)
    rh = (jnp.dot(state, rw1_ref[0:H, :])
          + jnp.dot(emb, rw1_ref[H:H + 4, :])
          + rb1_ref[...])
    rh = jnp.maximum(rh, 0.0)
    logits = jnp.dot(rh, rw2_ref[...]) + rb2_ref[...]
    mx = jnp.max(logits, axis=1, keepdims=True)
    cand = jnp.where(logits == mx,
                     jax.lax.broadcasted_iota(jnp.int32, (RBLK, T + 1), 1),
                     T + 1)
    action = jnp.min(cand, axis=1, keepdims=True)
    bins = jnp.where(done, T, action)
    oh9 = (bins == jax.lax.broadcasted_iota(jnp.int32, (RBLK, 128), 1)
           ).astype(jnp.float32)
    cum = jnp.dot(tril_ref[...], oh9)
    tot = jnp.sum(oh9, axis=0, keepdims=True)
    return bins, oh9, cum, tot


def _rank_from(oh9, cum, run_row):
    rank_f = jnp.sum(oh9 * (run_row + cum), axis=1, keepdims=True) - 1.0
    return rank_f.astype(jnp.int32)


def _emit_tables(cnt, triu8_ref, offs_out, cumb_out):
    nb = jnp.floor((cnt + (MBLK - 1)) * (1.0 / MBLK))
    cumb = jnp.dot(nb, triu8_ref[...])
    offs = (cumb - nb) * MBLK
    lane = jax.lax.broadcasted_iota(jnp.int32, (1, 128), 1)
    offs_out[...] = jnp.where(lane < T, offs, float(DUMP)).astype(jnp.int32)
    cumb_out[...] = cumb.astype(jnp.int32)


def _k1_body(x_ref, w_in_ref, b_in_ref, emb_ref, rw1_ref, rb1_ref, rw2_ref,
             rb2_ref, t0_ref, tril_ref, triu8_ref,
             state_out, bins_out, rank_out, offs_out, cumb_out, run_ref):
    i = pl.program_id(0)

    @pl.when(i == 0)
    def _():
        run_ref[...] = jnp.zeros((8, 128), jnp.float32)

    state = jnp.dot(x_ref[...], w_in_ref[...]) + b_in_ref[...]
    state_out[...] = state
    done = jnp.zeros((RBLK, 1), jnp.bool_)
    bins, oh9, cum, tot = _router(state, t0_ref[...], done, emb_ref, rw1_ref,
                                  rb1_ref, rw2_ref, rb2_ref, tril_ref)
    bins_out[...] = bins
    run_row = run_ref[0:1, :]
    rank_out[...] = _rank_from(oh9, cum, run_row)
    run_ref[0:1, :] = run_row + tot

    @pl.when(i == NRBLK - 1)
    def _():
        _emit_tables(run_row + tot, triu8_ref, offs_out, cumb_out)


def _k3_body(g1_ref, bins1_ref, emb_ref, rw1_ref, rb1_ref, rw2_ref, rb2_ref,
             tril_ref, triu8_ref,
             bins_out, rank_out, offs_out, cumb_out, run_ref):
    i = pl.program_id(0)

    @pl.when(i == 0)
    def _():
        run_ref[...] = jnp.zeros((8, 128), jnp.float32)

    bins1 = bins1_ref[...]
    done = bins1 == T
    bins, oh9, cum, tot = _router(g1_ref[...], bins1, done, emb_ref, rw1_ref,
                                  rb1_ref, rw2_ref, rb2_ref, tril_ref)
    bins_out[...] = bins
    run_row = run_ref[0:1, :]
    rank_out[...] = _rank_from(oh9, cum, run_row)
    run_ref[0:1, :] = run_row + tot

    @pl.when(i == NRBLK - 1)
    def _():
        _emit_tables(run_row + tot, triu8_ref, offs_out, cumb_out)


def _k5_body(state1_ref, g1_ref, bins1_ref, g2_ref, bins2_ref, lng_ref,
             lnb_ref, tw_ref, tb_ref, out_ref):
    s2 = jnp.where(bins1_ref[...] < T, g1_ref[...], state1_ref[...])
    state = jnp.where(bins2_ref[...] < T, g2_ref[...], s2)
    mu = jnp.mean(state, axis=1, keepdims=True)
    var = jnp.mean((state - mu) ** 2, axis=1, keepdims=True)
    normed = (state - mu) / jnp.sqrt(var + 1e-5) * lng_ref[...] + lnb_ref[...]
    out_ref[...] = jnp.dot(normed, tw_ref[...]) + tb_ref[...]


def _mlp_body(cumb_ref, oi_ref, xin_ref, w1_ref, b1_ref, w2_ref, b2_ref,
              o_ref):
    del cumb_ref, oi_ref
    h1 = jnp.maximum(jnp.dot(xin_ref[...], w1_ref[0, 0]) + b1_ref[0, 0], 0.0)
    o_ref[...] = jnp.maximum(jnp.dot(h1, w2_ref[0, 0]) + b2_ref[0, 0],
                             0.0) * 1.01


def _whole(shape):
    return pl.BlockSpec(shape, lambda i: tuple(0 for _ in shape))


@functools.lru_cache(maxsize=1)
def _sc_mesh():
    return plsc.VectorSubcoreMesh(core_axis_name="c", subcore_axis_name="s")


@functools.lru_cache(maxsize=1)
def _sc_params():
    cp = pltpu.CompilerParams()
    if "needs_layout_passes" in pltpu.CompilerParams.__dataclass_fields__:
        cp = dataclasses.replace(cp, needs_layout_passes=False)
    return cp


def _sc_scatter(bins2d, rank2d, offs16, state):
    @functools.partial(
        pl.kernel,
        out_type=(jax.ShapeDtypeStruct((PAD_ROWS, H), jnp.float32),
                  jax.ShapeDtypeStruct((NW, 2, 128), jnp.int32)),
        mesh=_sc_mesh(),
        scratch_types=[pltpu.VMEM((2, 128), jnp.int32),
                       pltpu.VMEM((2, 128), jnp.int32),
                       pltpu.VMEM((2, 128), jnp.int32),
                       pltpu.VMEM((16,), jnp.int32),
                       pltpu.VMEM((TPW, H), jnp.float32),
                       pltpu.SemaphoreType.DMA,
                       pltpu.SemaphoreType.DMA,
                       pltpu.SemaphoreType.DMA],
        compiler_params=_sc_params(),
    )
    def scat(bins_hbm, rank_hbm, offs_hbm, state_hbm, pad_out, pos_out,
             bins_v, rank_v, pos_v, offs_v, rows_v, sem, sem_i, sem_o):
        wid = lax.axis_index("s") * 2 + lax.axis_index("c")
        base = wid * TPW
        cp = pltpu.async_copy(state_hbm.at[pl.ds(base, TPW)], rows_v, sem)
        cb = pltpu.async_copy(bins_hbm.at[pl.ds(wid * 2, 2)], bins_v, sem_i)
        cr = pltpu.async_copy(rank_hbm.at[pl.ds(wid * 2, 2)], rank_v, sem_i)
        co = pltpu.async_copy(offs_hbm, offs_v, sem_i)
        cb.wait()
        cr.wait()
        co.wait()
        for cj in range(2):
            for j in range(8):
                sl = pl.ds(j * 16, 16)
                b = bins_v[cj, sl]
                r = rank_v[cj, sl]
                off = plsc.load_gather(offs_v, [b])
                pos_v[cj, sl] = jnp.where(b >= T, DUMP, off + r)
        cp.wait()
        w0 = pltpu.async_copy(pos_v, pos_out.at[wid], sem_o)
        w1 = pltpu.async_copy(rows_v.at[pl.ds(0, 128)],
                              pad_out.at[pos_v.at[0]], sem_o)
        w2 = pltpu.async_copy(rows_v.at[pl.ds(128, 128)],
                              pad_out.at[pos_v.at[1]], sem_o)
        w0.wait()
        w1.wait()
        w2.wait()

    return scat(bins2d, rank2d, offs16, state)


def _sc_gather(pos, mlp_out):
    @functools.partial(
        pl.kernel,
        out_type=jax.ShapeDtypeStruct((N_TOK, H), jnp.float32),
        mesh=_sc_mesh(),
        scratch_types=[pltpu.VMEM((2, 128), jnp.int32),
                       pltpu.VMEM((TPW, H), jnp.float32),
                       pltpu.SemaphoreType.DMA],
    )
    def gath(pos_hbm, mlp_hbm, out_hbm, pos_v, rows_v, sem):
        wid = lax.axis_index("s") * 2 + lax.axis_index("c")
        base = wid * TPW
        pltpu.sync_copy(pos_hbm.at[wid], pos_v)
        cp0 = pltpu.async_copy(mlp_hbm.at[pos_v.at[0]],
                               rows_v.at[pl.ds(0, 128)], sem)
        cp1 = pltpu.async_copy(mlp_hbm.at[pos_v.at[1]],
                               rows_v.at[pl.ds(128, 128)], sem)
        cp0.wait()
        cp1.wait()
        pltpu.sync_copy(rows_v, out_hbm.at[pl.ds(base, TPW)])

    return gath(pos, mlp_out)


def _grouped_mlp(cumb8, oi, pad_buf, op_w1, op_b1, op_w2, op_b2):
    def _wmap(g, cumb_ref, oi_ref):
        e = jnp.int32(0)
        for t in range(T):
            e = e + (g >= cumb_ref[t]).astype(jnp.int32)
        e = jnp.minimum(e, T - 1)
        return (e, oi_ref[e], 0, 0)

    grid_spec = pltpu.PrefetchScalarGridSpec(
        num_scalar_prefetch=2,
        grid=(GRID_MLP,),
        in_specs=[
            pl.BlockSpec((MBLK, H), lambda g, cumb_ref, oi_ref: (g, 0)),
            pl.BlockSpec((1, 1, H, H), _wmap),
            pl.BlockSpec((1, 1, 1, H), _wmap),
            pl.BlockSpec((1, 1, H, H), _wmap),
            pl.BlockSpec((1, 1, 1, H), _wmap),
        ],
        out_specs=pl.BlockSpec((MBLK, H), lambda g, cumb_ref, oi_ref: (g, 0)),
    )
    return pl.pallas_call(
        _mlp_body,
        grid_spec=grid_spec,
        out_shape=jax.ShapeDtypeStruct((PAD_ROWS, H), jnp.float32),
    )(cumb8, oi, pad_buf, op_w1, op_b1.reshape(T, 3, 1, H), op_w2,
      op_b2.reshape(T, 3, 1, H))


def kernel(x, W_in, b_in, op_W1, op_b1, op_W2, op_b2, operator_logits,
           temper_embed, route_W1, route_b1, route_W2, route_b2, ln_g, ln_b,
           task_W, task_b, init_tempers):
    oi = jnp.argmax(operator_logits, axis=-1).astype(jnp.int32)
    t0 = init_tempers.astype(jnp.int32).reshape(N_TOK, 1)
    tril = jnp.tril(jnp.ones((RBLK, RBLK), jnp.float32))
    triu = jnp.triu(jnp.ones((128, 128), jnp.float32))
    triu8 = jnp.where(jnp.arange(128)[:, None] < T, triu, 0.0)
    rb1 = route_b1.reshape(1, H)
    rb2 = route_b2.reshape(1, T + 1)

    tok_i32 = pl.BlockSpec((RBLK, 1), lambda i: (i, 0))
    tok_f32 = pl.BlockSpec((RBLK, H), lambda i: (i, 0))
    row128 = pl.BlockSpec((1, 128), lambda i: (0, 0))
    common_w = [_whole((T, 4)), _whole((H + 4, H)), _whole((1, H)),
                _whole((H, T + 1)), _whole((1, T + 1))]
    table_outs = [jax.ShapeDtypeStruct((1, 128), jnp.int32),
                  jax.ShapeDtypeStruct((1, 128), jnp.int32)]
    tok_outs = [jax.ShapeDtypeStruct((N_TOK, 1), jnp.int32),
                jax.ShapeDtypeStruct((N_TOK, 1), jnp.int32)]

    state1, bins1, rank1, offs1, cumb1 = pl.pallas_call(
        _k1_body,
        grid=(NRBLK,),
        in_specs=[pl.BlockSpec((RBLK, D_IN), lambda i: (i, 0)),
                  _whole((D_IN, H)), _whole((1, H))] + common_w
                 + [tok_i32, _whole((RBLK, RBLK)), _whole((128, 128))],
        out_specs=[tok_f32, tok_i32, tok_i32, row128, row128],
        out_shape=[jax.ShapeDtypeStruct((N_TOK, H), jnp.float32)] + tok_outs
                  + table_outs,
        scratch_shapes=[pltpu.VMEM((8, 128), jnp.float32)],
    )(x, W_in, b_in.reshape(1, H), temper_embed, route_W1, rb1, route_W2,
      rb2, t0, tril, triu8)

    def dispatch_round(source, bins, rank, offs, cumb):
        pad_buf, pos = _sc_scatter(bins.reshape(NW * 2, 128),
                                   rank.reshape(NW * 2, 128),
                                   offs[0, :16], source)
        mlp_out = _grouped_mlp(cumb[0, :T], oi, pad_buf, op_W1, op_b1,
                               op_W2, op_b2)
        return _sc_gather(pos, mlp_out)

    g1 = dispatch_round(state1, bins1, rank1, offs1, cumb1)

    bins2, rank2, offs2, cumb2 = pl.pallas_call(
        _k3_body,
        grid=(NRBLK,),
        in_specs=[tok_f32, tok_i32] + common_w
                 + [_whole((RBLK, RBLK)), _whole((128, 128))],
        out_specs=[tok_i32, tok_i32, row128, row128],
        out_shape=tok_outs + table_outs,
        scratch_shapes=[pltpu.VMEM((8, 128), jnp.float32)],
    )(g1, bins1, temper_embed, route_W1, rb1, route_W2, rb2, tril, triu8)

    g2 = dispatch_round(g1, bins2, rank2, offs2, cumb2)

    out = pl.pallas_call(
        _k5_body,
        grid=(NRBLK,),
        in_specs=[tok_f32, tok_f32, tok_i32, tok_f32, tok_i32,
                  _whole((1, H)), _whole((1, H)), _whole((H, 10)),
                  _whole((1, 10))],
        out_specs=pl.BlockSpec((RBLK, 10), lambda i: (i, 0)),
        out_shape=jax.ShapeDtypeStruct((N_TOK, 10), jnp.float32),
    )(state1, g1, bins1, g2, bins2, ln_g.reshape(1, H), ln_b.reshape(1, H),
      task_W, task_b.reshape(1, 10))
    return out
